# Initial kernel scaffold; baseline (speedup 1.0000x reference)
import jax
import jax.numpy as jnp
from jax.experimental import pallas as pl


def kernel(node_features, edge_index, edge_features, W_msg, b_msg, W_apply, b_apply):
    raise NotImplementedError("write your pallas kernel here")


# speedup vs baseline: 3.1104x; 3.1104x over previous
import functools

import jax
import jax.numpy as jnp
from jax import lax
from jax.experimental import pallas as pl
from jax.experimental.pallas import tpu as pltpu
from jax.experimental.pallas import tpu_sc as plsc

NC = 2
NS = 16
LANES = 16


def _matmul_bias(x, w, b, block_rows):
    rows, k = x.shape
    m = w.shape[1]

    def body(x_ref, w_ref, b_ref, o_ref):
        o_ref[...] = (
            jnp.dot(x_ref[...], w_ref[...], preferred_element_type=jnp.float32)
            + b_ref[...]
        )

    return pl.pallas_call(
        body,
        out_shape=jax.ShapeDtypeStruct((rows, m), jnp.float32),
        grid=(rows // block_rows,),
        in_specs=[
            pl.BlockSpec((block_rows, k), lambda i: (i, 0)),
            pl.BlockSpec((k, m), lambda i: (0, 0)),
            pl.BlockSpec((1, m), lambda i: (0, 0)),
        ],
        out_specs=pl.BlockSpec((block_rows, m), lambda i: (i, 0)),
    )(x, w, b.reshape(1, m))


def _apply_layer(x, parts, wa_top, wa_bot, b, block_rows):
    n, d = x.shape
    m = wa_top.shape[1]

    def body(x_ref, h_ref, wt_ref, wb_ref, b_ref, o_ref):
        h = h_ref[0] + h_ref[1]
        acc = jnp.dot(x_ref[...], wt_ref[...], preferred_element_type=jnp.float32)
        acc += jnp.dot(h, wb_ref[...], preferred_element_type=jnp.float32)
        o_ref[...] = jnp.maximum(acc + b_ref[...], 0.0)

    return pl.pallas_call(
        body,
        out_shape=jax.ShapeDtypeStruct((n, m), jnp.float32),
        grid=(n // block_rows,),
        in_specs=[
            pl.BlockSpec((block_rows, d), lambda i: (i, 0)),
            pl.BlockSpec((NC, block_rows, m), lambda i: (0, i, 0)),
            pl.BlockSpec((d, m), lambda i: (0, 0)),
            pl.BlockSpec((m, m), lambda i: (0, 0)),
            pl.BlockSpec((1, m), lambda i: (0, 0)),
        ],
        out_specs=pl.BlockSpec((block_rows, m), lambda i: (i, 0)),
    )(x, parts, wa_top, wa_bot, b.reshape(1, m))


def _sc_edge_kernel(n_nodes, n_edges, do, ch):
    n_workers = NC * NS
    epw = n_edges // n_workers
    nchunks = epw // ch
    rows_per_tile = n_nodes // NS

    mesh = plsc.VectorSubcoreMesh(
        core_axis_name="c", subcore_axis_name="s", num_cores=NC, num_subcores=NS
    )

    @functools.partial(
        pl.kernel,
        out_type=jax.ShapeDtypeStruct((NC, n_nodes, do), jnp.float32),
        mesh=mesh,
        scratch_types=[
            pltpu.VMEM((nchunks, ch), jnp.int32),
            pltpu.VMEM((ch,), jnp.int32),
            pltpu.VMEM((ch,), jnp.int32),
            pltpu.VMEM((ch, do), jnp.float32),
            pltpu.VMEM((ch, do), jnp.float32),
            pltpu.VMEM_SHARED((n_nodes, do), jnp.float32),
            pltpu.SemaphoreType.DMA,
        ],
    )
    def body(p_hbm, q_hbm, idx_hbm, out_hbm,
             idx_v, src_v, dst_v, p_v, q_v, acc, sem):
        cid = lax.axis_index("c")
        sid = lax.axis_index("s")
        wid = cid * NS + sid

        def zrow(j, c2):
            for l in range(do // LANES):
                q_v[j, pl.ds(l * LANES, LANES)] = jnp.zeros((LANES,), jnp.float32)
            return c2

        lax.fori_loop(0, ch, zrow, 0)
        r0 = sid * rows_per_tile
        zch = 8 * (ch // 8)
        nfull = rows_per_tile // zch
        rem = rows_per_tile - nfull * zch
        for zi in range(nfull):
            pltpu.sync_copy(q_v.at[pl.ds(0, zch)],
                            acc.at[pl.ds(r0 + zi * zch, zch)])
        if rem:
            pltpu.sync_copy(q_v.at[pl.ds(0, rem)],
                            acc.at[pl.ds(r0 + nfull * zch, rem)])

        crow = wid * nchunks
        pltpu.sync_copy(idx_hbm.at[wid], idx_v)
        plsc.subcore_barrier()

        def chunk(i, carry):
            for g in range(ch // LANES):
                s = pl.ds(g * LANES, LANES)
                packed = idx_v[i, s]
                src_v[s] = lax.bitwise_and(packed, 0xFFFF)
                dst_v[s] = lax.shift_right_logical(packed, 16)
            pltpu.async_copy(p_hbm.at[src_v], p_v, sem).wait()
            pltpu.sync_copy(q_hbm.at[crow + i], q_v)

            def row(j, c2):
                for l in range(do // LANES):
                    s = pl.ds(l * LANES, LANES)
                    p_v[j, s] = jnp.maximum(p_v[j, s] + q_v[j, s], 0.0)
                return c2

            lax.fori_loop(0, ch, row, 0)
            pltpu.sync_copy(p_v, acc.at[dst_v], add=True)
            return carry

        lax.fori_loop(0, nchunks, chunk, 0)

        plsc.subcore_barrier()
        pltpu.sync_copy(
            acc.at[pl.ds(r0, rows_per_tile)],
            out_hbm.at[cid, pl.ds(r0, rows_per_tile)],
        )

    return body


CH = 80


def kernel(node_features, edge_index, edge_features, W_msg, b_msg,
           W_apply, b_apply):
    n, d = node_features.shape
    e = edge_features.shape[0]
    do = W_msg.shape[1]

    npad = ((n + NS * 8 - 1) // (NS * 8)) * (NS * 8)
    x_pad = jnp.concatenate(
        [node_features, jnp.zeros((npad - n, d), jnp.float32)], axis=0)

    w_msg_top = W_msg[:d]
    w_msg_bot = W_msg[d:]
    wa_top = W_apply[:d]
    wa_bot = W_apply[d:]

    p = _matmul_bias(x_pad, w_msg_top, jnp.zeros((do,), jnp.float32),
                     block_rows=1264)
    q = _matmul_bias(edge_features, w_msg_bot, b_msg, block_rows=16000)

    nw = NC * NS
    packed = (edge_index[0] | (edge_index[1] << 16)).reshape(
        nw, e // (nw * CH), CH)

    sc = _sc_edge_kernel(npad, e, do, CH)
    parts = sc(p, q.reshape(e // CH, CH, do), packed)

    out = _apply_layer(x_pad, parts, wa_top, wa_bot, b_apply,
                       block_rows=1264)
    return out[:n]


# speedup vs baseline: 4.3126x; 1.3865x over previous
import functools

import jax
import jax.numpy as jnp
from jax import lax
from jax.experimental import pallas as pl
from jax.experimental.pallas import tpu as pltpu
from jax.experimental.pallas import tpu_sc as plsc

NC = 2
NS = 16
LANES = 16


def _matmul_bias(x, w, b, block_rows):
    rows, k = x.shape
    m = w.shape[1]

    def body(x_ref, w_ref, b_ref, o_ref):
        o_ref[...] = (
            jnp.dot(x_ref[...], w_ref[...], preferred_element_type=jnp.float32)
            + b_ref[...]
        )

    return pl.pallas_call(
        body,
        out_shape=jax.ShapeDtypeStruct((rows, m), jnp.float32),
        grid=(rows // block_rows,),
        in_specs=[
            pl.BlockSpec((block_rows, k), lambda i: (i, 0)),
            pl.BlockSpec((k, m), lambda i: (0, 0)),
            pl.BlockSpec((1, m), lambda i: (0, 0)),
        ],
        out_specs=pl.BlockSpec((block_rows, m), lambda i: (i, 0)),
    )(x, w, b.reshape(1, m))


def _apply_layer(x, parts, wa_top, wa_bot, b, block_rows):
    n, d = x.shape
    m = wa_top.shape[1]

    def body(x_ref, h_ref, wt_ref, wb_ref, b_ref, o_ref):
        h = h_ref[0] + h_ref[1]
        acc = jnp.dot(x_ref[...], wt_ref[...], preferred_element_type=jnp.float32)
        acc += jnp.dot(h, wb_ref[...], preferred_element_type=jnp.float32)
        o_ref[...] = jnp.maximum(acc + b_ref[...], 0.0)

    return pl.pallas_call(
        body,
        out_shape=jax.ShapeDtypeStruct((n, m), jnp.float32),
        grid=(n // block_rows,),
        in_specs=[
            pl.BlockSpec((block_rows, d), lambda i: (i, 0)),
            pl.BlockSpec((NC, block_rows, m), lambda i: (0, i, 0)),
            pl.BlockSpec((d, m), lambda i: (0, 0)),
            pl.BlockSpec((m, m), lambda i: (0, 0)),
            pl.BlockSpec((1, m), lambda i: (0, 0)),
        ],
        out_specs=pl.BlockSpec((block_rows, m), lambda i: (i, 0)),
    )(x, parts, wa_top, wa_bot, b.reshape(1, m))


def _sc_edge_kernel(n_nodes, n_edges, do, ch):
    n_workers = NC * NS
    epw = n_edges // n_workers
    nchunks = epw // ch
    rows_per_tile = n_nodes // NS

    mesh = plsc.VectorSubcoreMesh(
        core_axis_name="c", subcore_axis_name="s", num_cores=NC, num_subcores=NS
    )

    assert nchunks % 2 == 1

    @functools.partial(
        pl.kernel,
        out_type=jax.ShapeDtypeStruct((NC, n_nodes, do), jnp.float32),
        mesh=mesh,
        scratch_types=[
            pltpu.VMEM((nchunks, ch), jnp.int32),
            pltpu.VMEM((ch,), jnp.int32),
            pltpu.VMEM((ch,), jnp.int32),
            pltpu.VMEM((ch,), jnp.int32),
            pltpu.VMEM((ch,), jnp.int32),
            pltpu.VMEM((ch, do), jnp.float32),
            pltpu.VMEM((ch, do), jnp.float32),
            pltpu.VMEM((ch, do), jnp.float32),
            pltpu.VMEM_SHARED((n_nodes, do), jnp.float32),
            pltpu.SemaphoreType.DMA,
            pltpu.SemaphoreType.DMA,
            pltpu.SemaphoreType.DMA,
            pltpu.SemaphoreType.DMA,
            pltpu.SemaphoreType.DMA,
        ],
    )
    def body(p_hbm, q_hbm, idx_hbm, out_hbm,
             idx_v, src0, src1, dst0, dst1, pv0, pv1, qv0, acc,
             sg0, sg1, sq0, ss0, ss1):
        cid = lax.axis_index("c")
        sid = lax.axis_index("s")
        wid = cid * NS + sid
        src = (src0, src1)
        dst = (dst0, dst1)
        p_v = (pv0, pv1)
        ss = (ss0, ss1)
        sg = (sg0, sg1)

        def zrow(j, c2):
            for l in range(do // LANES):
                qv0[j, pl.ds(l * LANES, LANES)] = jnp.zeros((LANES,), jnp.float32)
            return c2

        lax.fori_loop(0, ch, zrow, 0)
        r0 = sid * rows_per_tile
        zch = 8 * (ch // 8)
        nfull = rows_per_tile // zch
        rem = rows_per_tile - nfull * zch
        for zi in range(nfull):
            pltpu.sync_copy(qv0.at[pl.ds(0, zch)],
                            acc.at[pl.ds(r0 + zi * zch, zch)])
        if rem:
            pltpu.sync_copy(qv0.at[pl.ds(0, rem)],
                            acc.at[pl.ds(r0 + nfull * zch, rem)])

        crow = wid * nchunks
        pltpu.sync_copy(idx_hbm.at[wid], idx_v)
        plsc.subcore_barrier()

        def unpack(i, b):
            for g in range(ch // LANES):
                s = pl.ds(g * LANES, LANES)
                packed = idx_v[i, s]
                src[b][s] = lax.bitwise_and(packed, 0xFFFF)
                dst[b][s] = lax.shift_right_logical(packed, 16)

        def issue_gather(b):
            pltpu.async_copy(p_hbm.at[src[b]], p_v[b], sg[b])

        def wait_gather(b):
            pltpu.make_async_copy(p_hbm.at[src[b]], p_v[b], sg[b]).wait()

        def issue_qload(i):
            pltpu.async_copy(q_hbm.at[crow + i], qv0, sq0)

        def wait_qload():
            pltpu.make_async_copy(q_hbm.at[crow], qv0, sq0).wait()

        def compute(b):
            def row(j, c2):
                for l in range(do // LANES):
                    s = pl.ds(l * LANES, LANES)
                    p_v[b][j, s] = jnp.maximum(
                        p_v[b][j, s] + qv0[j, s], 0.0)
                return c2

            lax.fori_loop(0, ch, row, 0)

        def issue_scatter(b):
            pltpu.async_copy(p_v[b], acc.at[dst[b]], ss[b], add=True)

        def wait_scatter(b):
            pltpu.make_async_copy(p_v[b], acc.at[dst[b]], ss[b]).wait()

        unpack(0, 0)
        issue_gather(0)
        issue_qload(0)

        def step(t, carry):
            for k in range(2):
                i = 2 * t + k
                if k == 0:
                    @pl.when(t > 0)
                    def _():
                        wait_scatter(1)
                else:
                    wait_scatter(0)
                unpack(i + 1, 1 - k)
                issue_gather(1 - k)
                wait_gather(k)
                wait_qload()
                compute(k)
                issue_qload(i + 1)
                issue_scatter(k)
            return carry

        lax.fori_loop(0, (nchunks - 1) // 2, step, 0)

        wait_scatter(1)
        wait_gather(0)
        wait_qload()
        compute(0)
        issue_scatter(0)
        wait_scatter(0)

        plsc.subcore_barrier()
        pltpu.sync_copy(
            acc.at[pl.ds(r0, rows_per_tile)],
            out_hbm.at[cid, pl.ds(r0, rows_per_tile)],
        )

    return body


CH = 80


def kernel(node_features, edge_index, edge_features, W_msg, b_msg,
           W_apply, b_apply):
    n, d = node_features.shape
    e = edge_features.shape[0]
    do = W_msg.shape[1]

    npad = ((n + NS * 8 - 1) // (NS * 8)) * (NS * 8)
    x_pad = jnp.concatenate(
        [node_features, jnp.zeros((npad - n, d), jnp.float32)], axis=0)

    w_msg_top = W_msg[:d]
    w_msg_bot = W_msg[d:]
    wa_top = W_apply[:d]
    wa_bot = W_apply[d:]

    p = _matmul_bias(x_pad, w_msg_top, jnp.zeros((do,), jnp.float32),
                     block_rows=1264)
    q = _matmul_bias(edge_features, w_msg_bot, b_msg, block_rows=16000)

    nw = NC * NS
    packed = (edge_index[0] | (edge_index[1] << 16)).reshape(
        nw, e // (nw * CH), CH)

    sc = _sc_edge_kernel(npad, e, do, CH)
    parts = sc(p, q.reshape(e // CH, CH, do), packed)

    out = _apply_layer(x_pad, parts, wa_top, wa_bot, b_apply,
                       block_rows=1264)
    return out[:n]


# speedup vs baseline: 4.4122x; 1.0231x over previous
import functools

import jax
import jax.numpy as jnp
from jax import lax
from jax.experimental import pallas as pl
from jax.experimental.pallas import tpu as pltpu
from jax.experimental.pallas import tpu_sc as plsc

NC = 2
NS = 16
LANES = 16


def _matmul_bias(x, w, b, block_rows):
    rows, k = x.shape
    m = w.shape[1]

    def body(x_ref, w_ref, b_ref, o_ref):
        o_ref[...] = (
            jnp.dot(x_ref[...], w_ref[...], preferred_element_type=jnp.float32)
            + b_ref[...]
        )

    return pl.pallas_call(
        body,
        out_shape=jax.ShapeDtypeStruct((rows, m), jnp.float32),
        grid=(rows // block_rows,),
        in_specs=[
            pl.BlockSpec((block_rows, k), lambda i: (i, 0)),
            pl.BlockSpec((k, m), lambda i: (0, 0)),
            pl.BlockSpec((1, m), lambda i: (0, 0)),
        ],
        out_specs=pl.BlockSpec((block_rows, m), lambda i: (i, 0)),
    )(x, w, b.reshape(1, m))


def _matmul_bias_3d(x, w, b, ch, blk_chunks):
    rows, k = x.shape
    m = w.shape[1]
    block_rows = blk_chunks * ch

    def body(x_ref, w_ref, b_ref, o_ref):
        res = (
            jnp.dot(x_ref[...], w_ref[...], preferred_element_type=jnp.float32)
            + b_ref[...]
        )
        o_ref[...] = res.reshape(blk_chunks, ch, m)

    return pl.pallas_call(
        body,
        out_shape=jax.ShapeDtypeStruct((rows // ch, ch, m), jnp.float32),
        grid=(rows // block_rows,),
        in_specs=[
            pl.BlockSpec((block_rows, k), lambda i: (i, 0)),
            pl.BlockSpec((k, m), lambda i: (0, 0)),
            pl.BlockSpec((1, m), lambda i: (0, 0)),
        ],
        out_specs=pl.BlockSpec((blk_chunks, ch, m), lambda i: (i, 0, 0)),
    )(x, w, b.reshape(1, m))


def _apply_layer(x, parts, wa_top, wa_bot, b, block_rows):
    n, d = x.shape
    m = wa_top.shape[1]

    def body(x_ref, h_ref, wt_ref, wb_ref, b_ref, o_ref):
        h = h_ref[0] + h_ref[1]
        acc = jnp.dot(x_ref[...], wt_ref[...], preferred_element_type=jnp.float32)
        acc += jnp.dot(h, wb_ref[...], preferred_element_type=jnp.float32)
        o_ref[...] = jnp.maximum(acc + b_ref[...], 0.0)

    return pl.pallas_call(
        body,
        out_shape=jax.ShapeDtypeStruct((n, m), jnp.float32),
        grid=(n // block_rows,),
        in_specs=[
            pl.BlockSpec((block_rows, d), lambda i: (i, 0)),
            pl.BlockSpec((NC, block_rows, m), lambda i: (0, i, 0)),
            pl.BlockSpec((d, m), lambda i: (0, 0)),
            pl.BlockSpec((m, m), lambda i: (0, 0)),
            pl.BlockSpec((1, m), lambda i: (0, 0)),
        ],
        out_specs=pl.BlockSpec((block_rows, m), lambda i: (i, 0)),
    )(x, parts, wa_top, wa_bot, b.reshape(1, m))


def _sc_edge_kernel(n_nodes, n_edges, do, ch):
    n_workers = NC * NS
    epw = n_edges // n_workers
    nchunks = epw // ch
    rows_per_tile = n_nodes // NS

    mesh = plsc.VectorSubcoreMesh(
        core_axis_name="c", subcore_axis_name="s", num_cores=NC, num_subcores=NS
    )

    assert nchunks % 2 == 1

    @functools.partial(
        pl.kernel,
        out_type=jax.ShapeDtypeStruct((NC, n_nodes, do), jnp.float32),
        mesh=mesh,
        scratch_types=[
            pltpu.VMEM((nchunks, ch), jnp.int32),
            pltpu.VMEM((ch,), jnp.int32),
            pltpu.VMEM((ch,), jnp.int32),
            pltpu.VMEM((ch,), jnp.int32),
            pltpu.VMEM((ch,), jnp.int32),
            pltpu.VMEM((ch, do), jnp.float32),
            pltpu.VMEM((ch, do), jnp.float32),
            pltpu.VMEM((ch, do), jnp.float32),
            pltpu.VMEM_SHARED((n_nodes, do), jnp.float32),
            pltpu.SemaphoreType.DMA,
            pltpu.SemaphoreType.DMA,
            pltpu.SemaphoreType.DMA,
            pltpu.SemaphoreType.DMA,
            pltpu.SemaphoreType.DMA,
        ],
    )
    def body(p_hbm, q_hbm, idx_hbm, out_hbm,
             idx_v, src0, src1, dst0, dst1, pv0, pv1, qv0, acc,
             sg0, sg1, sq0, ss0, ss1):
        cid = lax.axis_index("c")
        sid = lax.axis_index("s")
        wid = cid * NS + sid
        src = (src0, src1)
        dst = (dst0, dst1)
        p_v = (pv0, pv1)
        ss = (ss0, ss1)
        sg = (sg0, sg1)

        def zrow(j, c2):
            for l in range(do // LANES):
                qv0[j, pl.ds(l * LANES, LANES)] = jnp.zeros((LANES,), jnp.float32)
            return c2

        lax.fori_loop(0, ch, zrow, 0)
        r0 = sid * rows_per_tile
        zch = 8 * (ch // 8)
        nfull = rows_per_tile // zch
        rem = rows_per_tile - nfull * zch
        for zi in range(nfull):
            pltpu.sync_copy(qv0.at[pl.ds(0, zch)],
                            acc.at[pl.ds(r0 + zi * zch, zch)])
        if rem:
            pltpu.sync_copy(qv0.at[pl.ds(0, rem)],
                            acc.at[pl.ds(r0 + nfull * zch, rem)])

        crow = wid * nchunks
        pltpu.sync_copy(idx_hbm.at[wid], idx_v)
        plsc.subcore_barrier()

        def unpack(i, b):
            for g in range(ch // LANES):
                s = pl.ds(g * LANES, LANES)
                packed = idx_v[i, s]
                src[b][s] = lax.bitwise_and(packed, 0xFFFF)
                dst[b][s] = lax.shift_right_logical(packed, 16)

        def issue_gather(b):
            pltpu.async_copy(p_hbm.at[src[b]], p_v[b], sg[b])

        def wait_gather(b):
            pltpu.make_async_copy(p_hbm.at[src[b]], p_v[b], sg[b]).wait()

        def issue_qload(i):
            pltpu.async_copy(q_hbm.at[crow + i], qv0, sq0)

        def wait_qload():
            pltpu.make_async_copy(q_hbm.at[crow], qv0, sq0).wait()

        def compute(b):
            def row(j, c2):
                for l in range(do // LANES):
                    s = pl.ds(l * LANES, LANES)
                    p_v[b][j, s] = jnp.maximum(
                        p_v[b][j, s] + qv0[j, s], 0.0)
                return c2

            lax.fori_loop(0, ch, row, 0)

        def issue_scatter(b):
            pltpu.async_copy(p_v[b], acc.at[dst[b]], ss[b], add=True)

        def wait_scatter(b):
            pltpu.make_async_copy(p_v[b], acc.at[dst[b]], ss[b]).wait()

        unpack(0, 0)
        issue_gather(0)
        issue_qload(0)

        def step(t, carry):
            for k in range(2):
                i = 2 * t + k
                if k == 0:
                    @pl.when(t > 0)
                    def _():
                        wait_scatter(1)
                else:
                    wait_scatter(0)
                unpack(i + 1, 1 - k)
                issue_gather(1 - k)
                wait_gather(k)
                wait_qload()
                compute(k)
                issue_qload(i + 1)
                issue_scatter(k)
            return carry

        lax.fori_loop(0, (nchunks - 1) // 2, step, 0)

        wait_scatter(1)
        wait_gather(0)
        wait_qload()
        compute(0)
        issue_scatter(0)
        wait_scatter(0)

        plsc.subcore_barrier()
        pltpu.sync_copy(
            acc.at[pl.ds(r0, rows_per_tile)],
            out_hbm.at[cid, pl.ds(r0, rows_per_tile)],
        )

    return body


CH = 80


def kernel(node_features, edge_index, edge_features, W_msg, b_msg,
           W_apply, b_apply):
    n, d = node_features.shape
    e = edge_features.shape[0]
    do = W_msg.shape[1]

    npad = ((n + NS * 8 - 1) // (NS * 8)) * (NS * 8)

    w_msg_top = W_msg[:d]
    w_msg_bot = W_msg[d:]
    wa_top = W_apply[:d]
    wa_bot = W_apply[d:]

    p = _matmul_bias(node_features, w_msg_top, jnp.zeros((do,), jnp.float32),
                     block_rows=2000)
    q = _matmul_bias_3d(edge_features, w_msg_bot, b_msg, CH, 200)

    nw = NC * NS
    packed = (edge_index[0] | (edge_index[1] << 16)).reshape(
        nw, e // (nw * CH), CH)

    sc = _sc_edge_kernel(npad, e, do, CH)
    parts = sc(p, q, packed)

    return _apply_layer(node_features, parts, wa_top, wa_bot, b_apply,
                        block_rows=2000)


# speedup vs baseline: 5.9213x; 1.3420x over previous
import functools

import jax
import jax.numpy as jnp
from jax import lax
from jax.experimental import pallas as pl
from jax.experimental.pallas import tpu as pltpu
from jax.experimental.pallas import tpu_sc as plsc

NC = 2
NS = 16
LANES = 16


def _matmul_bias(x, w, b, block_rows):
    rows, k = x.shape
    m = w.shape[1]

    def body(x_ref, w_ref, b_ref, o_ref):
        o_ref[...] = (
            jnp.dot(x_ref[...], w_ref[...], preferred_element_type=jnp.float32)
            + b_ref[...]
        )

    return pl.pallas_call(
        body,
        out_shape=jax.ShapeDtypeStruct((rows, m), jnp.float32),
        grid=(rows // block_rows,),
        in_specs=[
            pl.BlockSpec((block_rows, k), lambda i: (i, 0)),
            pl.BlockSpec((k, m), lambda i: (0, 0)),
            pl.BlockSpec((1, m), lambda i: (0, 0)),
        ],
        out_specs=pl.BlockSpec((block_rows, m), lambda i: (i, 0)),
    )(x, w, b.reshape(1, m))


def _matmul_bias_3d(xt, w, b, ch, blk_chunks):
    k, rows = xt.shape
    m = w.shape[1]
    block_rows = blk_chunks * ch

    def body(xt_ref, w_ref, b_ref, o_ref):
        res = lax.dot_general(
            xt_ref[...], w_ref[...], (((0,), (0,)), ((), ())),
            preferred_element_type=jnp.float32,
        ) + b_ref[...]
        o_ref[...] = res.reshape(blk_chunks, ch, m)

    return pl.pallas_call(
        body,
        out_shape=jax.ShapeDtypeStruct((rows // ch, ch, m), jnp.float32),
        grid=(rows // block_rows,),
        in_specs=[
            pl.BlockSpec((k, block_rows), lambda i: (0, i)),
            pl.BlockSpec((k, m), lambda i: (0, 0)),
            pl.BlockSpec((1, m), lambda i: (0, 0)),
        ],
        out_specs=pl.BlockSpec((blk_chunks, ch, m), lambda i: (i, 0, 0)),
    )(xt, w, b.reshape(1, m))


def _apply_layer(x, parts, wa_top, wa_bot, b, block_rows):
    n, d = x.shape
    m = wa_top.shape[1]

    def body(x_ref, h_ref, wt_ref, wb_ref, b_ref, o_ref):
        h = h_ref[0] + h_ref[1]
        acc = jnp.dot(x_ref[...], wt_ref[...], preferred_element_type=jnp.float32)
        acc += jnp.dot(h, wb_ref[...], preferred_element_type=jnp.float32)
        o_ref[...] = jnp.maximum(acc + b_ref[...], 0.0)

    return pl.pallas_call(
        body,
        out_shape=jax.ShapeDtypeStruct((n, m), jnp.float32),
        grid=(n // block_rows,),
        in_specs=[
            pl.BlockSpec((block_rows, d), lambda i: (i, 0)),
            pl.BlockSpec((NC, block_rows, m), lambda i: (0, i, 0)),
            pl.BlockSpec((d, m), lambda i: (0, 0)),
            pl.BlockSpec((m, m), lambda i: (0, 0)),
            pl.BlockSpec((1, m), lambda i: (0, 0)),
        ],
        out_specs=pl.BlockSpec((block_rows, m), lambda i: (i, 0)),
    )(x, parts, wa_top, wa_bot, b.reshape(1, m))


def _sc_edge_kernel(n_nodes, n_edges, do, ch):
    n_workers = NC * NS
    epw = n_edges // n_workers
    nchunks = epw // ch
    rows_per_tile = n_nodes // NS

    mesh = plsc.VectorSubcoreMesh(
        core_axis_name="c", subcore_axis_name="s", num_cores=NC, num_subcores=NS
    )

    assert nchunks % 2 == 1

    @functools.partial(
        pl.kernel,
        out_type=jax.ShapeDtypeStruct((NC, n_nodes, do), jnp.float32),
        mesh=mesh,
        scratch_types=[
            pltpu.VMEM((nchunks, ch), jnp.int32),
            pltpu.VMEM((ch,), jnp.int32),
            pltpu.VMEM((ch,), jnp.int32),
            pltpu.VMEM((ch,), jnp.int32),
            pltpu.VMEM((ch,), jnp.int32),
            pltpu.VMEM((ch, do), jnp.float32),
            pltpu.VMEM((ch, do), jnp.float32),
            pltpu.VMEM((ch, do), jnp.float32),
            pltpu.VMEM_SHARED((n_nodes, do), jnp.float32),
            pltpu.SemaphoreType.DMA,
            pltpu.SemaphoreType.DMA,
            pltpu.SemaphoreType.DMA,
            pltpu.SemaphoreType.DMA,
            pltpu.SemaphoreType.DMA,
        ],
    )
    def body(p_hbm, q_hbm, idx_hbm, out_hbm,
             idx_v, src0, src1, dst0, dst1, pv0, pv1, qv0, acc,
             sg0, sg1, sq0, ss0, ss1):
        cid = lax.axis_index("c")
        sid = lax.axis_index("s")
        wid = cid * NS + sid
        src = (src0, src1)
        dst = (dst0, dst1)
        p_v = (pv0, pv1)
        ss = (ss0, ss1)
        sg = (sg0, sg1)

        def zrow(j, c2):
            for l in range(do // LANES):
                qv0[j, pl.ds(l * LANES, LANES)] = jnp.zeros((LANES,), jnp.float32)
            return c2

        lax.fori_loop(0, ch, zrow, 0)
        r0 = sid * rows_per_tile
        zch = 8 * (ch // 8)
        nfull = rows_per_tile // zch
        rem = rows_per_tile - nfull * zch
        for zi in range(nfull):
            pltpu.sync_copy(qv0.at[pl.ds(0, zch)],
                            acc.at[pl.ds(r0 + zi * zch, zch)])
        if rem:
            pltpu.sync_copy(qv0.at[pl.ds(0, rem)],
                            acc.at[pl.ds(r0 + nfull * zch, rem)])

        crow = wid * nchunks
        pltpu.sync_copy(idx_hbm.at[wid], idx_v)
        plsc.subcore_barrier()

        def unpack(i, b):
            for g in range(ch // LANES):
                s = pl.ds(g * LANES, LANES)
                packed = idx_v[i, s]
                src[b][s] = lax.bitwise_and(packed, 0xFFFF)
                dst[b][s] = lax.shift_right_logical(packed, 16)

        def issue_gather(b):
            pltpu.async_copy(p_hbm.at[src[b]], p_v[b], sg[b])

        def wait_gather(b):
            pltpu.make_async_copy(p_hbm.at[src[b]], p_v[b], sg[b]).wait()

        def issue_qload(i):
            pltpu.async_copy(q_hbm.at[crow + i], qv0, sq0)

        def wait_qload():
            pltpu.make_async_copy(q_hbm.at[crow], qv0, sq0).wait()

        def compute(b):
            def row(j, c2):
                for l in range(do // LANES):
                    s = pl.ds(l * LANES, LANES)
                    p_v[b][j, s] = jnp.maximum(
                        p_v[b][j, s] + qv0[j, s], 0.0)
                return c2

            lax.fori_loop(0, ch, row, 0)

        def issue_scatter(b):
            pltpu.async_copy(p_v[b], acc.at[dst[b]], ss[b], add=True)

        def wait_scatter(b):
            pltpu.make_async_copy(p_v[b], acc.at[dst[b]], ss[b]).wait()

        unpack(0, 0)
        issue_gather(0)
        issue_qload(0)

        def step(t, carry):
            for k in range(2):
                i = 2 * t + k
                if k == 0:
                    @pl.when(t > 0)
                    def _():
                        wait_scatter(1)
                else:
                    wait_scatter(0)
                unpack(i + 1, 1 - k)
                issue_gather(1 - k)
                wait_gather(k)
                wait_qload()
                compute(k)
                issue_qload(i + 1)
                issue_scatter(k)
            return carry

        lax.fori_loop(0, (nchunks - 1) // 2, step, 0)

        wait_scatter(1)
        wait_gather(0)
        wait_qload()
        compute(0)
        issue_scatter(0)
        wait_scatter(0)

        plsc.subcore_barrier()
        pltpu.sync_copy(
            acc.at[pl.ds(r0, rows_per_tile)],
            out_hbm.at[cid, pl.ds(r0, rows_per_tile)],
        )

    return body


CH = 80


def kernel(node_features, edge_index, edge_features, W_msg, b_msg,
           W_apply, b_apply):
    n, d = node_features.shape
    e = edge_features.shape[0]
    do = W_msg.shape[1]

    npad = ((n + NS * 8 - 1) // (NS * 8)) * (NS * 8)

    w_msg_top = W_msg[:d]
    w_msg_bot = W_msg[d:]
    wa_top = W_apply[:d]
    wa_bot = W_apply[d:]

    p = _matmul_bias(node_features, w_msg_top, jnp.zeros((do,), jnp.float32),
                     block_rows=2000)
    q = _matmul_bias_3d(edge_features.T, w_msg_bot, b_msg, CH, 200)

    nw = NC * NS
    packed = (edge_index[0] | (edge_index[1] << 16)).reshape(
        nw, e // (nw * CH), CH)

    sc = _sc_edge_kernel(npad, e, do, CH)
    parts = sc(p, q, packed)

    return _apply_layer(node_features, parts, wa_top, wa_bot, b_apply,
                        block_rows=2000)
